# baseline (speedup 1.0000x reference)
import jax
import jax.numpy as jnp
from jax.experimental import pallas as pl
from jax.experimental.pallas import tpu as pltpu

_F32 = jnp.float32
_BF16 = jnp.bfloat16


def _fused_kernel(x_ref, t1_ref, b1_ref, w3a_ref, w3b_ref, w3c_ref,
                  w3d_ref, w3e_ref, w3f_ref, b3_ref,
                  w5_ref, b5_ref, w6_ref, b6_ref, wo_ref, bo_ref, out_ref):
    M8 = x_ref.shape[0]
    nb = out_ref.shape[0]

    x = x_ref[...]
    L = M8 - 1
    t1 = t1_ref[...]

    xp0 = jnp.concatenate([x[0:L, 0:112], x[1:1 + L, 0:28]], axis=1)
    xp1 = jnp.concatenate([x[0:L, 28:112], x[1:1 + L, 0:56]], axis=1)
    xp2 = jnp.concatenate([x[0:L, 56:112], x[1:1 + L, 0:84]], axis=1)
    xp3 = jnp.concatenate([x[0:L, 84:112], x[1:1 + L, 0:112]], axis=1)
    c10 = jnp.dot(xp0, t1, preferred_element_type=_F32)
    c11 = jnp.dot(xp1, t1, preferred_element_type=_F32)
    c12 = jnp.dot(xp2, t1, preferred_element_type=_F32)
    c13 = jnp.dot(xp3, t1, preferred_element_type=_F32)

    b1 = b1_ref[...]
    pe = jnp.maximum(c10, c11) + b1
    po = jnp.maximum(c12, c13) + b1
    se = jnp.maximum(jnp.maximum(pe[:, 0:128], pe[:, 128:256]), 0.0)
    so = jnp.maximum(jnp.maximum(po[:, 0:128], po[:, 128:256]), 0.0)
    se = se.astype(_BF16)
    so = so.astype(_BF16)
    eo = jnp.concatenate([se, so], axis=1)

    L3 = M8 - 3
    c3e = (jnp.dot(eo[0:L3, :], w3a_ref[...], preferred_element_type=_F32)
           + jnp.dot(eo[1:1 + L3, :], w3b_ref[...], preferred_element_type=_F32)
           + jnp.dot(se[2:2 + L3, :], w3c_ref[...], preferred_element_type=_F32))
    c3o = (jnp.dot(so[0:L3, :], w3f_ref[...], preferred_element_type=_F32)
           + jnp.dot(eo[1:1 + L3, :], w3d_ref[...], preferred_element_type=_F32)
           + jnp.dot(eo[2:2 + L3, :], w3e_ref[...], preferred_element_type=_F32))

    b3 = b3_ref[...]
    p4 = jnp.maximum(c3e, c3o) + b3
    a4 = jnp.maximum(jnp.maximum(p4[:, 0:128], p4[:, 128:256]), 0.0)
    a4 = a4.astype(_BF16)

    a4 = jnp.concatenate([a4, jnp.zeros((3, 128), _BF16)], axis=0)
    r3 = a4.reshape(nb, 8, 128)
    xc5 = jnp.concatenate([r3[:, dy, :] for dy in range(5)], axis=1)
    feats = jnp.maximum(
        jnp.dot(xc5, w5_ref[...], preferred_element_type=_F32)
        + b5_ref[...], 0.0)

    h = jnp.maximum(
        jnp.dot(feats.astype(_BF16), w6_ref[...], preferred_element_type=_F32)
        + b6_ref[...], 0.0)
    out_ref[...] = (jnp.dot(h.astype(_BF16), wo_ref[...],
                            preferred_element_type=_F32)
                    + bo_ref[...]).astype(out_ref.dtype)


def kernel(img, T1, B1, T3, B3, T5, B5, W6p, B6p, WOp, BOp, *, block_batch=256):
    B = img.shape[0]
    nb = block_batch
    Bp = ((B + nb - 1) // nb) * nb

    x = img.reshape(B, 28, 28)
    x = jnp.pad(x, ((0, Bp - B), (2, 2), (0, 0)))
    x = x.reshape(Bp * 8, 112).astype(_BF16)

    T1f = T1[:, 2:30, :].reshape(140, 256)
    W3a = T3[0:2].reshape(256, 256)
    W3b = T3[2:4].reshape(256, 256)
    W3c = T3[4]
    W3f = T3[0]
    W3d = T3[1:3].reshape(256, 256)
    W3e = T3[3:5].reshape(256, 256)
    W5f = T5.reshape(640, 128)

    weights = (T1f, B1, W3a, W3b, W3c, W3d, W3e, W3f, B3,
               W5f, B5, W6p, B6p, WOp, BOp)
    w_specs = [pl.BlockSpec(t.shape, lambda i, n=t.ndim: (0,) * n)
               for t in weights]
    grid = (Bp // nb,)

    out = pl.pallas_call(
        _fused_kernel,
        out_shape=jax.ShapeDtypeStruct((Bp, 128), jnp.float32),
        grid=grid,
        in_specs=[pl.BlockSpec((nb * 8, 112), lambda i: (i, 0))] + w_specs,
        out_specs=pl.BlockSpec((nb, 128), lambda i: (i, 0)),
        compiler_params=pltpu.CompilerParams(
            dimension_semantics=("parallel",),
            vmem_limit_bytes=64 * 1024 * 1024),
    )(x, *weights)

    return out[:B, :10]


# speedup vs baseline: 4.2166x; 4.2166x over previous
import jax
import jax.numpy as jnp
from jax.experimental import pallas as pl
from jax.experimental.pallas import tpu as pltpu

_F32 = jnp.float32
_BF16 = jnp.bfloat16


def _fused_kernel(x_ref, ma_ref, mb_ref, t1_ref, b1_ref, w3a_ref, w3b_ref,
                  w3c_ref, w3d_ref, w3e_ref, w3f_ref, b3_ref,
                  w5_ref, b5_ref, w6_ref, b6_ref, wo_ref, bo_ref, out_ref):
    M7 = x_ref.shape[0]
    nb = out_ref.shape[0]

    u = x_ref[...].astype(_BF16)
    t1 = t1_ref[...]

    uprev = jnp.concatenate([jnp.zeros((1, 56), _BF16),
                             u[0:M7 - 1, 56:112]], axis=0)
    unext = jnp.concatenate([u[1:M7, 0:112],
                             jnp.zeros((1, 112), _BF16)], axis=0)
    uprev = uprev * ma_ref[...]
    unext = unext * mb_ref[...]

    xp0 = jnp.concatenate([uprev, u[:, 0:84]], axis=1)
    xp1 = jnp.concatenate([uprev[:, 28:56], u[:, 0:112]], axis=1)
    xp2 = jnp.concatenate([u[:, 0:112], unext[:, 0:28]], axis=1)
    xp3 = jnp.concatenate([u[:, 28:112], unext[:, 0:56]], axis=1)
    c10 = jnp.dot(xp0, t1, preferred_element_type=_F32)
    c11 = jnp.dot(xp1, t1, preferred_element_type=_F32)
    c12 = jnp.dot(xp2, t1, preferred_element_type=_F32)
    c13 = jnp.dot(xp3, t1, preferred_element_type=_F32)

    b1 = b1_ref[...]
    pe = jnp.maximum(c10, c11) + b1
    po = jnp.maximum(c12, c13) + b1
    se = jnp.maximum(jnp.maximum(pe[:, 0:128], pe[:, 128:256]), 0.0)
    so = jnp.maximum(jnp.maximum(po[:, 0:128], po[:, 128:256]), 0.0)
    se = se.astype(_BF16)
    so = so.astype(_BF16)
    eo = jnp.concatenate([se, so], axis=1)

    L3 = M7 - 2
    c3e = (jnp.dot(eo[0:L3, :], w3a_ref[...], preferred_element_type=_F32)
           + jnp.dot(eo[1:1 + L3, :], w3b_ref[...], preferred_element_type=_F32)
           + jnp.dot(se[2:2 + L3, :], w3c_ref[...], preferred_element_type=_F32))
    c3o = (jnp.dot(so[0:L3, :], w3f_ref[...], preferred_element_type=_F32)
           + jnp.dot(eo[1:1 + L3, :], w3d_ref[...], preferred_element_type=_F32)
           + jnp.dot(eo[2:2 + L3, :], w3e_ref[...], preferred_element_type=_F32))

    b3 = b3_ref[...]
    p4 = jnp.maximum(c3e, c3o) + b3
    a4 = jnp.maximum(jnp.maximum(p4[:, 0:128], p4[:, 128:256]), 0.0)
    a4 = a4.astype(_BF16)

    a4 = jnp.concatenate([a4, jnp.zeros((2, 128), _BF16)], axis=0)
    r3 = a4.reshape(nb, 7, 128)
    xc5 = jnp.concatenate([r3[:, dy, :] for dy in range(5)], axis=1)
    feats = jnp.maximum(
        jnp.dot(xc5, w5_ref[...], preferred_element_type=_F32)
        + b5_ref[...], 0.0)

    h = jnp.maximum(
        jnp.dot(feats.astype(_BF16), w6_ref[...], preferred_element_type=_F32)
        + b6_ref[...], 0.0)
    out_ref[...] = (jnp.dot(h.astype(_BF16), wo_ref[...],
                            preferred_element_type=_F32)
                    + bo_ref[...]).astype(out_ref.dtype)


def kernel(img, T1, B1, T3, B3, T5, B5, W6p, B6p, WOp, BOp, *, block_batch=256):
    B = img.shape[0]
    nb = block_batch
    Bp = ((B + nb - 1) // nb) * nb

    x = img.reshape(B * 7, 112)
    if Bp != B:
        x = jnp.pad(x, ((0, (Bp - B) * 7), (0, 0)))

    mA = jnp.tile(jnp.concatenate([jnp.zeros((1, 56), _BF16),
                                   jnp.ones((6, 56), _BF16)], axis=0),
                  (nb, 1))
    mB = jnp.tile(jnp.concatenate([jnp.ones((6, 112), _BF16),
                                   jnp.zeros((1, 112), _BF16)], axis=0),
                  (nb, 1))

    T1f = T1[:, 2:30, :].reshape(140, 256)
    W3a = T3[0:2].reshape(256, 256)
    W3b = T3[2:4].reshape(256, 256)
    W3c = T3[4]
    W3f = T3[0]
    W3d = T3[1:3].reshape(256, 256)
    W3e = T3[3:5].reshape(256, 256)
    W5f = T5.reshape(640, 128)

    weights = (mA, mB, T1f, B1, W3a, W3b, W3c, W3d, W3e, W3f, B3,
               W5f, B5, W6p, B6p, WOp, BOp)
    w_specs = [pl.BlockSpec(t.shape, lambda i, n=t.ndim: (0,) * n)
               for t in weights]
    grid = (Bp // nb,)

    out = pl.pallas_call(
        _fused_kernel,
        out_shape=jax.ShapeDtypeStruct((Bp, 128), jnp.float32),
        grid=grid,
        in_specs=[pl.BlockSpec((nb * 7, 112), lambda i: (i, 0))] + w_specs,
        out_specs=pl.BlockSpec((nb, 128), lambda i: (i, 0)),
        compiler_params=pltpu.CompilerParams(
            dimension_semantics=("parallel",),
            vmem_limit_bytes=64 * 1024 * 1024),
    )(x, *weights)

    return out[:B, :10]


# speedup vs baseline: 4.2238x; 1.0017x over previous
import jax
import jax.numpy as jnp
from jax.experimental import pallas as pl
from jax.experimental.pallas import tpu as pltpu

_F32 = jnp.float32
_BF16 = jnp.bfloat16


def _fused_kernel(x_ref, ma_ref, mb_ref, t1_ref, b1_ref, w3a_ref, w3b_ref,
                  w3c_ref, w3d_ref, w3e_ref, w3f_ref, b3_ref,
                  w5_ref, b5_ref, w6_ref, b6_ref, wo_ref, bo_ref, out_ref):
    M7 = x_ref.shape[0]
    nb = out_ref.shape[0]

    u = x_ref[...].astype(_BF16)
    t1 = t1_ref[...]

    uprev = jnp.concatenate([jnp.zeros((1, 56), _BF16),
                             u[0:M7 - 1, 56:112]], axis=0)
    unext = jnp.concatenate([u[1:M7, 0:112],
                             jnp.zeros((1, 112), _BF16)], axis=0)
    uprev = uprev * ma_ref[...]
    unext = unext * mb_ref[...]

    xp0 = jnp.concatenate([uprev, u[:, 0:84]], axis=1)
    xp1 = jnp.concatenate([uprev[:, 28:56], u[:, 0:112]], axis=1)
    xp2 = jnp.concatenate([u[:, 0:112], unext[:, 0:28]], axis=1)
    xp3 = jnp.concatenate([u[:, 28:112], unext[:, 0:56]], axis=1)
    c10 = jnp.dot(xp0, t1, preferred_element_type=_F32)
    c11 = jnp.dot(xp1, t1, preferred_element_type=_F32)
    c12 = jnp.dot(xp2, t1, preferred_element_type=_F32)
    c13 = jnp.dot(xp3, t1, preferred_element_type=_F32)

    b1 = b1_ref[...]
    pe = jnp.maximum(c10, c11) + b1
    po = jnp.maximum(c12, c13) + b1
    se = jnp.maximum(jnp.maximum(pe[:, 0:128], pe[:, 128:256]), 0.0)
    so = jnp.maximum(jnp.maximum(po[:, 0:128], po[:, 128:256]), 0.0)
    se = se.astype(_BF16)
    so = so.astype(_BF16)
    eo = jnp.concatenate([se, so], axis=1)

    L3 = M7 - 2
    c3e = (jnp.dot(eo[0:L3, :], w3a_ref[...], preferred_element_type=_F32)
           + jnp.dot(eo[1:1 + L3, :], w3b_ref[...], preferred_element_type=_F32)
           + jnp.dot(se[2:2 + L3, :], w3c_ref[...], preferred_element_type=_F32))
    c3o = (jnp.dot(so[0:L3, :], w3f_ref[...], preferred_element_type=_F32)
           + jnp.dot(eo[1:1 + L3, :], w3d_ref[...], preferred_element_type=_F32)
           + jnp.dot(eo[2:2 + L3, :], w3e_ref[...], preferred_element_type=_F32))

    b3 = b3_ref[...]
    p4 = jnp.maximum(c3e, c3o) + b3
    a4 = jnp.maximum(jnp.maximum(p4[:, 0:128], p4[:, 128:256]), 0.0)
    a4 = a4.astype(_BF16)

    a4 = jnp.concatenate([a4, jnp.zeros((2, 128), _BF16)], axis=0)
    r3 = a4.reshape(nb, 7, 128)
    xc5 = jnp.concatenate([r3[:, dy, :] for dy in range(5)], axis=1)
    feats = jnp.maximum(
        jnp.dot(xc5, w5_ref[...], preferred_element_type=_F32)
        + b5_ref[...], 0.0)

    h = jnp.maximum(
        jnp.dot(feats.astype(_BF16), w6_ref[...], preferred_element_type=_F32)
        + b6_ref[...], 0.0)
    logits = (jnp.dot(h.astype(_BF16), wo_ref[...],
                      preferred_element_type=_F32) + bo_ref[...])
    out_ref[...] = logits[:, 0:10].astype(out_ref.dtype)


def kernel(img, T1, B1, T3, B3, T5, B5, W6p, B6p, WOp, BOp, *, block_batch=256):
    B = img.shape[0]
    nb = block_batch
    Bp = ((B + nb - 1) // nb) * nb

    x = img.reshape(B * 7, 112)
    if Bp != B:
        x = jnp.pad(x, ((0, (Bp - B) * 7), (0, 0)))

    mA = jnp.tile(jnp.concatenate([jnp.zeros((1, 56), _BF16),
                                   jnp.ones((6, 56), _BF16)], axis=0),
                  (nb, 1))
    mB = jnp.tile(jnp.concatenate([jnp.ones((6, 112), _BF16),
                                   jnp.zeros((1, 112), _BF16)], axis=0),
                  (nb, 1))

    T1f = T1[:, 2:30, :].reshape(140, 256)
    W3a = T3[0:2].reshape(256, 256)
    W3b = T3[2:4].reshape(256, 256)
    W3c = T3[4]
    W3f = T3[0]
    W3d = T3[1:3].reshape(256, 256)
    W3e = T3[3:5].reshape(256, 256)
    W5f = T5.reshape(640, 128)

    weights = (mA, mB, T1f, B1, W3a, W3b, W3c, W3d, W3e, W3f, B3,
               W5f, B5, W6p, B6p, WOp, BOp)
    w_specs = [pl.BlockSpec(t.shape, lambda i, n=t.ndim: (0,) * n)
               for t in weights]
    grid = (Bp // nb,)

    out = pl.pallas_call(
        _fused_kernel,
        out_shape=jax.ShapeDtypeStruct((Bp, 10), jnp.float32),
        grid=grid,
        in_specs=[pl.BlockSpec((nb * 7, 112), lambda i: (i, 0))] + w_specs,
        out_specs=pl.BlockSpec((nb, 10), lambda i: (i, 0)),
        compiler_params=pltpu.CompilerParams(
            dimension_semantics=("parallel",),
            vmem_limit_bytes=64 * 1024 * 1024),
    )(x, *weights)

    return out[:B]


# speedup vs baseline: 4.2286x; 1.0012x over previous
import jax
import jax.numpy as jnp
from jax.experimental import pallas as pl
from jax.experimental.pallas import tpu as pltpu

_F32 = jnp.float32
_BF16 = jnp.bfloat16


def _fused_kernel(x_ref, ma_ref, mb_ref, t1_ref, b1_ref, w3a_ref, w3b_ref,
                  w3c_ref, w3d_ref, w3e_ref, w3f_ref, b3_ref,
                  w5_ref, b5_ref, w6_ref, b6_ref, wo_ref, bo_ref, out_ref):
    M7 = x_ref.shape[0]
    nb = out_ref.shape[0]

    u = x_ref[...].astype(_BF16)
    t1 = t1_ref[...]

    uprev = jnp.concatenate([jnp.zeros((1, 56), _BF16),
                             u[0:M7 - 1, 56:112]], axis=0)
    unext = jnp.concatenate([u[1:M7, 0:112],
                             jnp.zeros((1, 112), _BF16)], axis=0)
    uprev = uprev * ma_ref[...]
    unext = unext * mb_ref[...]

    xp0 = jnp.concatenate([uprev, u[:, 0:84]], axis=1)
    xp1 = jnp.concatenate([uprev[:, 28:56], u[:, 0:112]], axis=1)
    xp2 = jnp.concatenate([u[:, 0:112], unext[:, 0:28]], axis=1)
    xp3 = jnp.concatenate([u[:, 28:112], unext[:, 0:56]], axis=1)
    c10 = jnp.dot(xp0, t1, preferred_element_type=_F32)
    c11 = jnp.dot(xp1, t1, preferred_element_type=_F32)
    c12 = jnp.dot(xp2, t1, preferred_element_type=_F32)
    c13 = jnp.dot(xp3, t1, preferred_element_type=_F32)

    b1 = b1_ref[...]
    pe = jnp.maximum(c10, c11) + b1
    po = jnp.maximum(c12, c13) + b1
    se = jnp.maximum(jnp.maximum(pe[:, 0:128], pe[:, 128:256]), 0.0)
    so = jnp.maximum(jnp.maximum(po[:, 0:128], po[:, 128:256]), 0.0)
    se = se.astype(_BF16)
    so = so.astype(_BF16)
    eo = jnp.concatenate([se, so], axis=1)

    L3 = M7 - 2
    c3e = (jnp.dot(eo[0:L3, :], w3a_ref[...], preferred_element_type=_F32)
           + jnp.dot(eo[1:1 + L3, :], w3b_ref[...], preferred_element_type=_F32)
           + jnp.dot(se[2:2 + L3, :], w3c_ref[...], preferred_element_type=_F32))
    c3o = (jnp.dot(so[0:L3, :], w3f_ref[...], preferred_element_type=_F32)
           + jnp.dot(eo[1:1 + L3, :], w3d_ref[...], preferred_element_type=_F32)
           + jnp.dot(eo[2:2 + L3, :], w3e_ref[...], preferred_element_type=_F32))

    b3 = b3_ref[...]
    p4 = jnp.maximum(c3e, c3o) + b3
    a4 = jnp.maximum(jnp.maximum(p4[:, 0:128], p4[:, 128:256]), 0.0)
    a4 = a4.astype(_BF16)

    a4 = jnp.concatenate([a4, jnp.zeros((2, 128), _BF16)], axis=0)
    r3 = a4.reshape(nb, 7, 128)
    xc5 = jnp.concatenate([r3[:, dy, :] for dy in range(5)], axis=1)
    feats = jnp.maximum(
        jnp.dot(xc5, w5_ref[...], preferred_element_type=_F32)
        + b5_ref[...], 0.0)

    h = jnp.maximum(
        jnp.dot(feats.astype(_BF16), w6_ref[...], preferred_element_type=_F32)
        + b6_ref[...], 0.0)
    logits = (jnp.dot(h.astype(_BF16), wo_ref[...],
                      preferred_element_type=_F32) + bo_ref[...])
    out_ref[...] = logits[:, 0:10].astype(out_ref.dtype)


def kernel(img, T1, B1, T3, B3, T5, B5, W6p, B6p, WOp, BOp, *, block_batch=256):
    B = img.shape[0]
    nb = block_batch
    Bp = ((B + nb - 1) // nb) * nb

    x = img.reshape(B * 7, 112)
    if Bp != B:
        x = jnp.pad(x, ((0, (Bp - B) * 7), (0, 0)))

    mA = jnp.tile(jnp.concatenate([jnp.zeros((1, 56), _BF16),
                                   jnp.ones((6, 56), _BF16)], axis=0),
                  (nb, 1))
    mB = jnp.tile(jnp.concatenate([jnp.ones((6, 112), _BF16),
                                   jnp.zeros((1, 112), _BF16)], axis=0),
                  (nb, 1))

    T1f = T1[:, 2:30, :].reshape(140, 256)
    W3a = T3[0:2].reshape(256, 256)
    W3b = T3[2:4].reshape(256, 256)
    W3c = T3[4]
    W3f = T3[0]
    W3d = T3[1:3].reshape(256, 256)
    W3e = T3[3:5].reshape(256, 256)
    W5f = T5.reshape(640, 128)

    weights = (mA, mB, T1f, B1, W3a, W3b, W3c, W3d, W3e, W3f, B3,
               W5f, B5, W6p, B6p, WOp, BOp)
    w_specs = [pl.BlockSpec(t.shape, lambda i, n=t.ndim: (0,) * n)
               for t in weights]
    grid = (Bp // nb,)

    out = pl.pallas_call(
        _fused_kernel,
        out_shape=jax.ShapeDtypeStruct((Bp, 10), jnp.float32),
        grid=grid,
        in_specs=[pl.BlockSpec((nb * 7, 112), lambda i: (i, 0))] + w_specs,
        out_specs=pl.BlockSpec((nb, 10), lambda i: (i, 0)),
        compiler_params=pltpu.CompilerParams(
            dimension_semantics=("arbitrary",),
            vmem_limit_bytes=64 * 1024 * 1024),
    )(x, *weights)

    return out[:B]


# speedup vs baseline: 4.6966x; 1.1107x over previous
import jax
import jax.numpy as jnp
from jax.experimental import pallas as pl
from jax.experimental.pallas import tpu as pltpu

_F32 = jnp.float32
_BF16 = jnp.bfloat16


def _fused_kernel(x_ref, t1_ref, b1_ref, w3a_ref, w3b_ref,
                  w3c_ref, w3d_ref, w3e_ref, w3f_ref, b3_ref,
                  w5_ref, b5_ref, w6_ref, b6_ref, wo_ref, bo_ref, out_ref):
    M7 = x_ref.shape[0]
    nb = out_ref.shape[0]

    u = x_ref[...].astype(_BF16)
    t1 = t1_ref[...]

    uprev = jnp.concatenate([jnp.zeros((1, 56), _BF16),
                             u[0:M7 - 1, 56:112]], axis=0)
    unext = jnp.concatenate([u[1:M7, 0:112],
                             jnp.zeros((1, 112), _BF16)], axis=0)
    ka = jax.lax.broadcasted_iota(jnp.int32, (M7, 56), 0) % 7
    kb = jax.lax.broadcasted_iota(jnp.int32, (M7, 112), 0) % 7
    uprev = jnp.where(ka == 0, jnp.bfloat16(0), uprev)
    unext = jnp.where(kb == 6, jnp.bfloat16(0), unext)

    xp0 = jnp.concatenate([uprev, u[:, 0:84]], axis=1)
    xp1 = jnp.concatenate([uprev[:, 28:56], u[:, 0:112]], axis=1)
    xp2 = jnp.concatenate([u[:, 0:112], unext[:, 0:28]], axis=1)
    xp3 = jnp.concatenate([u[:, 28:112], unext[:, 0:56]], axis=1)
    c10 = jnp.dot(xp0, t1, preferred_element_type=_F32)
    c11 = jnp.dot(xp1, t1, preferred_element_type=_F32)
    c12 = jnp.dot(xp2, t1, preferred_element_type=_F32)
    c13 = jnp.dot(xp3, t1, preferred_element_type=_F32)

    b1 = b1_ref[...]
    pe = jnp.maximum(c10, c11) + b1
    po = jnp.maximum(c12, c13) + b1
    se = jnp.maximum(jnp.maximum(pe[:, 0:128], pe[:, 128:256]), 0.0)
    so = jnp.maximum(jnp.maximum(po[:, 0:128], po[:, 128:256]), 0.0)
    se = se.astype(_BF16)
    so = so.astype(_BF16)
    eo = jnp.concatenate([se, so], axis=1)

    L3 = M7 - 2
    c3e = (jnp.dot(eo[0:L3, :], w3a_ref[...], preferred_element_type=_F32)
           + jnp.dot(eo[1:1 + L3, :], w3b_ref[...], preferred_element_type=_F32)
           + jnp.dot(se[2:2 + L3, :], w3c_ref[...], preferred_element_type=_F32))
    c3o = (jnp.dot(so[0:L3, :], w3f_ref[...], preferred_element_type=_F32)
           + jnp.dot(eo[1:1 + L3, :], w3d_ref[...], preferred_element_type=_F32)
           + jnp.dot(eo[2:2 + L3, :], w3e_ref[...], preferred_element_type=_F32))

    b3 = b3_ref[...]
    p4 = jnp.maximum(c3e, c3o) + b3
    a4 = jnp.maximum(jnp.maximum(p4[:, 0:128], p4[:, 128:256]), 0.0)
    a4 = a4.astype(_BF16)

    a4 = jnp.concatenate([a4, jnp.zeros((2, 128), _BF16)], axis=0)
    r3 = a4.reshape(nb, 7, 128)
    xc5 = jnp.concatenate([r3[:, dy, :] for dy in range(5)], axis=1)
    feats = jnp.maximum(
        jnp.dot(xc5, w5_ref[...], preferred_element_type=_F32)
        + b5_ref[...], 0.0)

    h = jnp.maximum(
        jnp.dot(feats.astype(_BF16), w6_ref[...], preferred_element_type=_F32)
        + b6_ref[...], 0.0)
    logits = (jnp.dot(h.astype(_BF16), wo_ref[...],
                      preferred_element_type=_F32) + bo_ref[...])
    out_ref[...] = logits[:, 0:10].astype(out_ref.dtype)


def kernel(img, T1, B1, T3, B3, T5, B5, W6p, B6p, WOp, BOp, *, block_batch=1024):
    B = img.shape[0]
    nb = block_batch
    Bp = ((B + nb - 1) // nb) * nb

    x = img.reshape(B * 7, 112)
    if Bp != B:
        x = jnp.pad(x, ((0, (Bp - B) * 7), (0, 0)))

    T1f = T1[:, 2:30, :].reshape(140, 256)
    W3a = T3[0:2].reshape(256, 256)
    W3b = T3[2:4].reshape(256, 256)
    W3c = T3[4]
    W3f = T3[0]
    W3d = T3[1:3].reshape(256, 256)
    W3e = T3[3:5].reshape(256, 256)
    W5f = T5.reshape(640, 128)

    weights = (T1f, B1, W3a, W3b, W3c, W3d, W3e, W3f, B3,
               W5f, B5, W6p, B6p, WOp, BOp)
    w_specs = [pl.BlockSpec(t.shape, lambda i, n=t.ndim: (0,) * n)
               for t in weights]
    grid = (Bp // nb,)

    out = pl.pallas_call(
        _fused_kernel,
        out_shape=jax.ShapeDtypeStruct((Bp, 10), jnp.float32),
        grid=grid,
        in_specs=[pl.BlockSpec((nb * 7, 112), lambda i: (i, 0))] + w_specs,
        out_specs=pl.BlockSpec((nb, 10), lambda i: (i, 0)),
        compiler_params=pltpu.CompilerParams(
            dimension_semantics=("parallel",),
            vmem_limit_bytes=64 * 1024 * 1024),
    )(x, *weights)

    return out[:B]
